# Initial kernel scaffold; baseline (speedup 1.0000x reference)
import jax
import jax.numpy as jnp
from jax.experimental import pallas as pl


def kernel(x):
    raise NotImplementedError("write your pallas kernel here")


# speedup vs baseline: 28.3544x; 28.3544x over previous
import functools

import jax
import jax.numpy as jnp
from jax import lax
from jax.experimental import pallas as pl

KNN = 16
TILE_P = 128


def _cubic_root_hi(r):
    c = jnp.ones_like(r)
    for _ in range(28):
        f = (4.0 * c * c - 3.0) * c - r
        fp = 12.0 * c * c - 3.0
        c = jnp.clip(c - f / jnp.maximum(fp, 1e-12), 0.5, 1.0)
    return c


def _cubic_root_lo(r):
    c = -jnp.ones_like(r)
    for _ in range(28):
        f = (4.0 * c * c - 3.0) * c - r
        fp = 12.0 * c * c - 3.0
        c = jnp.clip(c - f / jnp.maximum(fp, 1e-12), -1.0, -0.5)
    return c


def _ratio_from_moments(s, inv_k):
    mux = s[:, 0:1] * inv_k
    muy = s[:, 1:2] * inv_k
    muz = s[:, 2:3] * inv_k
    cxx = s[:, 3:4] * inv_k - mux * mux
    cxy = s[:, 4:5] * inv_k - mux * muy
    cxz = s[:, 5:6] * inv_k - mux * muz
    cyy = s[:, 7:8] * inv_k - muy * muy
    cyz = s[:, 8:9] * inv_k - muy * muz
    czz = s[:, 11:12] * inv_k - muz * muz

    q = (cxx + cyy + czz) * (1.0 / 3.0)
    axx = cxx - q
    ayy = cyy - q
    azz = czz - q
    p2 = (axx * axx + ayy * ayy + azz * azz
          + 2.0 * (cxy * cxy + cxz * cxz + cyz * cyz))
    p = jnp.sqrt(p2 * (1.0 / 6.0))
    inv_p = 1.0 / jnp.maximum(p, 1e-20)
    bxx = axx * inv_p
    byy = ayy * inv_p
    bzz = azz * inv_p
    bxy = cxy * inv_p
    bxz = cxz * inv_p
    byz = cyz * inv_p
    det_b = (bxx * (byy * bzz - byz * byz)
             - bxy * (bxy * bzz - byz * bxz)
             + bxz * (bxy * byz - byy * bxz))
    r = jnp.clip(0.5 * det_b, -1.0, 1.0)
    c_hi = _cubic_root_hi(r)
    c_lo = _cubic_root_lo(r)
    c_mid = -(c_hi + c_lo)
    lam0 = q + 2.0 * p * c_hi
    lam1 = q + 2.0 * p * c_mid
    return lam0 / lam1


def _body(xt_ref, feats_ref, out_ref, *, n_points, tile_p):
    t = pl.program_id(1)
    xt = xt_ref[0]
    xp = feats_ref[0, pl.ds(t * tile_p, tile_p), 0:3]

    sq = jnp.sum(xt * xt, axis=0, keepdims=True)
    sqp = jnp.sum(xp * xp, axis=1, keepdims=True)
    prod = lax.dot_general(xp.astype(jnp.bfloat16), xt.astype(jnp.bfloat16),
                           (((1,), (0,)), ((), ())),
                           preferred_element_type=jnp.float32)
    d = sqp + sq - 2.0 * prod

    iota = lax.broadcasted_iota(jnp.int32, (tile_p, n_points), 1)
    msum = jnp.zeros((tile_p, n_points), jnp.float32)
    for _ in range(KNN):
        m = jnp.min(d, axis=1, keepdims=True)
        eq = d == m
        fi = jnp.min(jnp.where(eq, iota, n_points), axis=1, keepdims=True)
        oh = iota == fi
        msum = jnp.where(oh, 1.0, msum)
        d = jnp.where(oh, jnp.inf, d)

    s = lax.dot_general(msum, feats_ref[0], (((1,), (0,)), ((), ())),
                        preferred_element_type=jnp.float32,
                        precision=lax.Precision.HIGHEST)
    out_ref[0] = _ratio_from_moments(s, 1.0 / KNN)


def kernel(x):
    x = x[..., :3]
    b, n, _ = x.shape
    feats = jnp.concatenate(
        [x, x[..., 0:1] * x, x[..., 1:2] * x, x[..., 2:3] * x], axis=-1)
    xt = jnp.swapaxes(x, 1, 2)
    nt = n // TILE_P
    out = pl.pallas_call(
        functools.partial(_body, n_points=n, tile_p=TILE_P),
        grid=(b, nt),
        in_specs=[
            pl.BlockSpec((1, 3, n), lambda bi, ti: (bi, 0, 0)),
            pl.BlockSpec((1, n, 12), lambda bi, ti: (bi, 0, 0)),
        ],
        out_specs=pl.BlockSpec((1, TILE_P, 1), lambda bi, ti: (bi, ti, 0)),
        out_shape=jax.ShapeDtypeStruct((b, n, 1), jnp.float32),
    )(xt, feats)
    return out[..., 0]


# speedup vs baseline: 57.6910x; 2.0346x over previous
import functools

import jax
import jax.numpy as jnp
from jax import lax
from jax.experimental import pallas as pl

KNN = 16
TILE_P = 128


def _cubic_root_hi(r):
    c = jnp.ones_like(r)
    for _ in range(28):
        f = (4.0 * c * c - 3.0) * c - r
        fp = 12.0 * c * c - 3.0
        c = jnp.clip(c - f / jnp.maximum(fp, 1e-12), 0.5, 1.0)
    return c


def _cubic_root_lo(r):
    c = -jnp.ones_like(r)
    for _ in range(28):
        f = (4.0 * c * c - 3.0) * c - r
        fp = 12.0 * c * c - 3.0
        c = jnp.clip(c - f / jnp.maximum(fp, 1e-12), -1.0, -0.5)
    return c


def _ratio_from_moments(s, inv_k):
    mux = s[:, 0:1] * inv_k
    muy = s[:, 1:2] * inv_k
    muz = s[:, 2:3] * inv_k
    cxx = s[:, 3:4] * inv_k - mux * mux
    cxy = s[:, 4:5] * inv_k - mux * muy
    cxz = s[:, 5:6] * inv_k - mux * muz
    cyy = s[:, 7:8] * inv_k - muy * muy
    cyz = s[:, 8:9] * inv_k - muy * muz
    czz = s[:, 11:12] * inv_k - muz * muz

    q = (cxx + cyy + czz) * (1.0 / 3.0)
    axx = cxx - q
    ayy = cyy - q
    azz = czz - q
    p2 = (axx * axx + ayy * ayy + azz * azz
          + 2.0 * (cxy * cxy + cxz * cxz + cyz * cyz))
    p = jnp.sqrt(p2 * (1.0 / 6.0))
    inv_p = 1.0 / jnp.maximum(p, 1e-20)
    bxx = axx * inv_p
    byy = ayy * inv_p
    bzz = azz * inv_p
    bxy = cxy * inv_p
    bxz = cxz * inv_p
    byz = cyz * inv_p
    det_b = (bxx * (byy * bzz - byz * byz)
             - bxy * (bxy * bzz - byz * bxz)
             + bxz * (bxy * byz - byy * bxz))
    r = jnp.clip(0.5 * det_b, -1.0, 1.0)
    c_hi = _cubic_root_hi(r)
    c_lo = _cubic_root_lo(r)
    c_mid = -(c_hi + c_lo)
    lam0 = q + 2.0 * p * c_hi
    lam1 = q + 2.0 * p * c_mid
    return lam0 / lam1


def _body(xt_ref, feats_ref, out_ref, *, n_points, tile_p):
    t = pl.program_id(1)
    xt = xt_ref[0]
    xp = feats_ref[0, pl.ds(t * tile_p, tile_p), 0:3]

    sq = jnp.sum(xt * xt, axis=0, keepdims=True)
    sqp = jnp.sum(xp * xp, axis=1, keepdims=True)
    prod = lax.dot_general(xp.astype(jnp.bfloat16), xt.astype(jnp.bfloat16),
                           (((1,), (0,)), ((), ())),
                           preferred_element_type=jnp.float32)
    d = sqp + sq - 2.0 * prod

    for _ in range(KNN):
        m = jnp.min(d, axis=1, keepdims=True)
        d = jnp.where(d == m, jnp.inf, d)
    msum = jnp.where(jnp.isinf(d), 1.0, 0.0)

    s = lax.dot_general(msum, feats_ref[0], (((1,), (0,)), ((), ())),
                        preferred_element_type=jnp.float32,
                        precision=lax.Precision.HIGHEST)
    out_ref[0] = _ratio_from_moments(s, 1.0 / KNN)


def kernel(x):
    x = x[..., :3]
    b, n, _ = x.shape
    feats = jnp.concatenate(
        [x, x[..., 0:1] * x, x[..., 1:2] * x, x[..., 2:3] * x], axis=-1)
    xt = jnp.swapaxes(x, 1, 2)
    nt = n // TILE_P
    out = pl.pallas_call(
        functools.partial(_body, n_points=n, tile_p=TILE_P),
        grid=(b, nt),
        in_specs=[
            pl.BlockSpec((1, 3, n), lambda bi, ti: (bi, 0, 0)),
            pl.BlockSpec((1, n, 12), lambda bi, ti: (bi, 0, 0)),
        ],
        out_specs=pl.BlockSpec((1, TILE_P, 1), lambda bi, ti: (bi, ti, 0)),
        out_shape=jax.ShapeDtypeStruct((b, n, 1), jnp.float32),
    )(xt, feats)
    return out[..., 0]


# speedup vs baseline: 77.5409x; 1.3441x over previous
import functools

import jax
import jax.numpy as jnp
from jax import lax
from jax.experimental import pallas as pl

KNN = 16
TILE_P = 128


def _cubic_root_hi(r):
    c = jnp.ones_like(r)
    for _ in range(28):
        f = (4.0 * c * c - 3.0) * c - r
        fp = 12.0 * c * c - 3.0
        c = jnp.clip(c - f / jnp.maximum(fp, 1e-12), 0.5, 1.0)
    return c


def _cubic_root_lo(r):
    c = -jnp.ones_like(r)
    for _ in range(28):
        f = (4.0 * c * c - 3.0) * c - r
        fp = 12.0 * c * c - 3.0
        c = jnp.clip(c - f / jnp.maximum(fp, 1e-12), -1.0, -0.5)
    return c


def _ratio_from_moments(s, inv_k):
    mux = s[0:1, :] * inv_k
    muy = s[1:2, :] * inv_k
    muz = s[2:3, :] * inv_k
    cxx = s[3:4, :] * inv_k - mux * mux
    cxy = s[4:5, :] * inv_k - mux * muy
    cxz = s[5:6, :] * inv_k - mux * muz
    cyy = s[7:8, :] * inv_k - muy * muy
    cyz = s[8:9, :] * inv_k - muy * muz
    czz = s[11:12, :] * inv_k - muz * muz

    q = (cxx + cyy + czz) * (1.0 / 3.0)
    axx = cxx - q
    ayy = cyy - q
    azz = czz - q
    p2 = (axx * axx + ayy * ayy + azz * azz
          + 2.0 * (cxy * cxy + cxz * cxz + cyz * cyz))
    p = jnp.sqrt(p2 * (1.0 / 6.0))
    inv_p = 1.0 / jnp.maximum(p, 1e-20)
    bxx = axx * inv_p
    byy = ayy * inv_p
    bzz = azz * inv_p
    bxy = cxy * inv_p
    bxz = cxz * inv_p
    byz = cyz * inv_p
    det_b = (bxx * (byy * bzz - byz * byz)
             - bxy * (bxy * bzz - byz * bxz)
             + bxz * (bxy * byz - byy * bxz))
    r = jnp.clip(0.5 * det_b, -1.0, 1.0)
    c_hi = _cubic_root_hi(r)
    c_lo = _cubic_root_lo(r)
    c_mid = -(c_hi + c_lo)
    lam0 = q + 2.0 * p * c_hi
    lam1 = q + 2.0 * p * c_mid
    return lam0 / lam1


def _body(xt_ref, feats_ref, out_ref, *, n_points, tile_p):
    t = pl.program_id(1)
    xt = xt_ref[0]
    xp = feats_ref[0, pl.ds(t * tile_p, tile_p), 0:3]

    sq = jnp.sum(xt * xt, axis=0, keepdims=True)
    sqp = jnp.sum(xp * xp, axis=1, keepdims=True)
    prod = lax.dot_general(xp.astype(jnp.bfloat16), xt.astype(jnp.bfloat16),
                           (((1,), (0,)), ((), ())),
                           preferred_element_type=jnp.float32)
    d = sqp + sq - 2.0 * prod

    for _ in range(KNN):
        m = jnp.min(d, axis=1, keepdims=True)
        d = jnp.where(d == m, jnp.inf, d)
    msum = jnp.where(jnp.isinf(d), 1.0, 0.0)

    s = lax.dot_general(feats_ref[0], msum, (((0,), (1,)), ((), ())),
                        preferred_element_type=jnp.float32,
                        precision=lax.Precision.HIGHEST)
    out_ref[0, 0] = _ratio_from_moments(s, 1.0 / KNN)


def kernel(x):
    x = x[..., :3]
    b, n, _ = x.shape
    feats = jnp.concatenate(
        [x, x[..., 0:1] * x, x[..., 1:2] * x, x[..., 2:3] * x], axis=-1)
    xt = jnp.swapaxes(x, 1, 2)
    nt = n // TILE_P
    out = pl.pallas_call(
        functools.partial(_body, n_points=n, tile_p=TILE_P),
        grid=(b, nt),
        in_specs=[
            pl.BlockSpec((1, 3, n), lambda bi, ti: (bi, 0, 0)),
            pl.BlockSpec((1, n, 12), lambda bi, ti: (bi, 0, 0)),
        ],
        out_specs=pl.BlockSpec((1, 1, 1, TILE_P), lambda bi, ti: (bi, ti, 0, 0)),
        out_shape=jax.ShapeDtypeStruct((b, nt, 1, TILE_P), jnp.float32),
    )(xt, feats)
    return out.reshape(b, n)
